# baseline (speedup 1.0000x reference)
import functools

import jax
import jax.numpy as jnp
from jax.experimental import pallas as pl
from jax.experimental.pallas import tpu as pltpu


def _gcn_kernel(scal_ref, a_hbm, f_ref, h0_ref, w_ref, out_ref, abuf, sems,
                *, rb, nbuf, nb):
    i = pl.program_id(0)

    def start(slot, blk):
        pltpu.make_async_copy(
            a_hbm.at[pl.ds(blk * rb, rb), :], abuf.at[slot], sems.at[slot]
        ).start()

    @pl.when(i == 0)
    def _():
        for s in range(nbuf - 1):
            start(s, s)

    slot = jax.lax.rem(i, nbuf)
    pltpu.make_async_copy(
        a_hbm.at[pl.ds(i * rb, rb), :], abuf.at[slot], sems.at[slot]
    ).wait()

    nxt = i + nbuf - 1

    @pl.when(nxt < nb)
    def _():
        start(jax.lax.rem(nxt, nbuf), nxt)

    alpha = scal_ref[0]
    theta = scal_ref[1]
    agg = jnp.dot(abuf[slot], f_ref[...], preferred_element_type=jnp.float32)
    support = (1.0 - alpha) * agg + alpha * h0_ref[...]
    lin = jnp.dot(support, w_ref[...], preferred_element_type=jnp.float32)
    out_ref[...] = jnp.maximum(theta * lin + (1.0 - theta) * support, 0.0)


def kernel(features, A, h0, W, lamda, alpha, l):
    B, N, D = features.shape
    theta = jnp.log(lamda / l + 1.0)
    scal = jnp.stack([jnp.float32(alpha), jnp.float32(theta)])
    f2 = features.reshape(N, D)
    h2 = h0.reshape(N, D)

    RB = 200
    NBUF = 6
    nb = N // RB
    out = pl.pallas_call(
        functools.partial(_gcn_kernel, rb=RB, nbuf=NBUF, nb=nb),
        grid=(nb,),
        in_specs=[
            pl.BlockSpec(memory_space=pltpu.SMEM),
            pl.BlockSpec(memory_space=pl.ANY),
            pl.BlockSpec((N, D), lambda i: (0, 0)),
            pl.BlockSpec((RB, D), lambda i: (i, 0)),
            pl.BlockSpec((D, D), lambda i: (0, 0)),
        ],
        out_specs=pl.BlockSpec((RB, D), lambda i: (i, 0)),
        out_shape=jax.ShapeDtypeStruct((N, D), jnp.float32),
        scratch_shapes=[
            pltpu.VMEM((NBUF, RB, N), jnp.float32),
            pltpu.SemaphoreType.DMA((NBUF,)),
        ],
    )(scal, A, f2, h2, W)
    return out.reshape(B, N, D)


# speedup vs baseline: 1.0313x; 1.0313x over previous
import functools

import jax
import jax.numpy as jnp
from jax.experimental import pallas as pl
from jax.experimental.pallas import tpu as pltpu


def _gcn_kernel(scal_ref, a_hbm, f_ref, h0_ref, w_ref, out_ref, abuf, sems,
                *, rb, nbuf, nb):
    i = pl.program_id(0)

    def start(slot, blk):
        pltpu.make_async_copy(
            a_hbm.at[pl.ds(blk * rb, rb), :], abuf.at[slot], sems.at[slot]
        ).start()

    @pl.when(i == 0)
    def _():
        for s in range(nbuf - 1):
            start(s, s)

    slot = jax.lax.rem(i, nbuf)
    pltpu.make_async_copy(
        a_hbm.at[pl.ds(i * rb, rb), :], abuf.at[slot], sems.at[slot]
    ).wait()

    nxt = i + nbuf - 1

    @pl.when(nxt < nb)
    def _():
        start(jax.lax.rem(nxt, nbuf), nxt)

    alpha = scal_ref[0]
    theta = scal_ref[1]
    agg = jnp.dot(abuf[slot], f_ref[...], preferred_element_type=jnp.float32)
    support = (1.0 - alpha) * agg + alpha * h0_ref[...]
    lin = jnp.dot(support, w_ref[...], preferred_element_type=jnp.float32)
    out_ref[...] = jnp.maximum(theta * lin + (1.0 - theta) * support, 0.0)


def kernel(features, A, h0, W, lamda, alpha, l):
    B, N, D = features.shape
    theta = jnp.log(lamda / l + 1.0)
    scal = jnp.stack([jnp.float32(alpha), jnp.float32(theta)])
    f2 = features.reshape(N, D)
    h2 = h0.reshape(N, D)

    RB = 200
    NBUF = 3
    nb = N // RB
    out = pl.pallas_call(
        functools.partial(_gcn_kernel, rb=RB, nbuf=NBUF, nb=nb),
        grid=(nb,),
        in_specs=[
            pl.BlockSpec(memory_space=pltpu.SMEM),
            pl.BlockSpec(memory_space=pl.ANY),
            pl.BlockSpec((N, D), lambda i: (0, 0)),
            pl.BlockSpec((RB, D), lambda i: (i, 0)),
            pl.BlockSpec((D, D), lambda i: (0, 0)),
        ],
        out_specs=pl.BlockSpec((RB, D), lambda i: (i, 0)),
        out_shape=jax.ShapeDtypeStruct((N, D), jnp.float32),
        scratch_shapes=[
            pltpu.VMEM((NBUF, RB, N), jnp.float32),
            pltpu.SemaphoreType.DMA((NBUF,)),
        ],
    )(scal, A, f2, h2, W)
    return out.reshape(B, N, D)
